# Initial kernel scaffold; baseline (speedup 1.0000x reference)
import jax
import jax.numpy as jnp
from jax.experimental import pallas as pl


def kernel(x, Wr, br, We, be):
    raise NotImplementedError("write your pallas kernel here")


# speedup vs baseline: 1.5067x; 1.5067x over previous
import functools

import jax
import jax.numpy as jnp
from jax import lax
from jax.experimental import pallas as pl
from jax.experimental.pallas import tpu as pltpu
from jax.experimental.pallas import tpu_sc as plsc

B, T, C = 2, 2048, 768
E = 8
N = B * T
M = 256
NB = N // M + (E - 1)
PAD_N = NB * M
WV = 128
NC, NS = 2, 16
NW = NC * NS
TPW = N // NW
G, GS = NW, TPW


def _router_body(x_ref, wr_ref, br_ref, p_ref, bexp_ref, w_ref, aux_ref):
    x = x_ref[...]
    logits = jnp.dot(x, wr_ref[...], preferred_element_type=jnp.float32)
    logits = logits + br_ref[...]
    m = jnp.max(logits, axis=1, keepdims=True)
    ex = jnp.exp(logits - m)
    inv_s = 1.0 / jnp.sum(ex, axis=1, keepdims=True)
    probs = ex * inv_s

    is_max = (logits == m).astype(jnp.float32)
    u8 = (lax.broadcasted_iota(jnp.int32, (E, E), 0)
          < lax.broadcasted_iota(jnp.int32, (E, E), 1)).astype(jnp.float32)
    before = jnp.dot(is_max, u8, preferred_element_type=jnp.float32)
    oh = is_max * (before == 0.0).astype(jnp.float32)

    imp = jnp.sum(probs, axis=0, keepdims=True) * (1.0 / N)
    loadv = jnp.sum(oh, axis=0, keepdims=True) * (1.0 / N)
    aux_ref[...] = jnp.sum(imp * loadv, keepdims=True)

    w_ref[...] = jnp.broadcast_to(inv_s, (N, WV))

    oh3 = oh.reshape(G, GS, E)
    ltri = (lax.broadcasted_iota(jnp.int32, (GS, GS), 0)
            > lax.broadcasted_iota(jnp.int32, (GS, GS), 1)).astype(jnp.float32)
    l3 = jnp.broadcast_to(ltri[None], (G, GS, GS))
    within = lax.dot_general(l3, oh3, (((2,), (1,)), ((0,), (0,))),
                             preferred_element_type=jnp.float32)
    ones3 = jnp.ones((G, 1, GS), jnp.float32)
    tot = lax.dot_general(ones3, oh3, (((2,), (1,)), ((0,), (0,))),
                          preferred_element_type=jnp.float32).reshape(G, E)
    lg = (lax.broadcasted_iota(jnp.int32, (G, G), 0)
          > lax.broadcasted_iota(jnp.int32, (G, G), 1)).astype(jnp.float32)
    gpre = jnp.dot(lg, tot, preferred_element_type=jnp.float32)

    counts = jnp.sum(tot, axis=0, keepdims=True)
    ci = counts.astype(jnp.int32)
    pc = ((ci + (M - 1)) // M) * M
    base = jnp.dot(pc.astype(jnp.float32), u8,
                   preferred_element_type=jnp.float32)

    slot = within + gpre[:, None, :] + base.reshape(1, 1, E)
    p_ref[...] = jnp.sum(oh3 * slot, axis=2).astype(jnp.int32)

    nbe = (base.astype(jnp.int32) + pc) // M
    bi = lax.broadcasted_iota(jnp.int32, (NB, E), 0)
    bexp = jnp.sum((bi >= jnp.broadcast_to(nbe, (NB, E))).astype(jnp.int32),
                   axis=1)
    bexp_ref[...] = jnp.minimum(bexp, E - 1).reshape(1, NB)


_router = pl.pallas_call(
    _router_body,
    out_shape=[
        jax.ShapeDtypeStruct((G, GS), jnp.int32),
        jax.ShapeDtypeStruct((1, NB), jnp.int32),
        jax.ShapeDtypeStruct((N, WV), jnp.float32),
        jax.ShapeDtypeStruct((1, 1), jnp.float32),
    ],
)


def _expert_body(bexp_ref, xs_ref, ws_ref, we_ref, be_ref, ys_ref):
    del bexp_ref
    xb = xs_ref[...].astype(jnp.bfloat16)
    wb = we_ref[0].astype(jnp.bfloat16)
    y = jnp.dot(xb, wb, preferred_element_type=jnp.float32)
    y = (y + be_ref[0]) * ws_ref[:, 0:1]
    ys_ref[...] = y


_expert_mm = pl.pallas_call(
    _expert_body,
    grid_spec=pltpu.PrefetchScalarGridSpec(
        num_scalar_prefetch=1,
        grid=(NB,),
        in_specs=[
            pl.BlockSpec((M, C), lambda i, bexp: (i, 0)),
            pl.BlockSpec((M, WV), lambda i, bexp: (i, 0)),
            pl.BlockSpec((1, C, C), lambda i, bexp: (bexp[i], 0, 0)),
            pl.BlockSpec((1, 1, C), lambda i, bexp: (bexp[i], 0, 0)),
        ],
        out_specs=pl.BlockSpec((M, C), lambda i, bexp: (i, 0)),
    ),
    out_shape=jax.ShapeDtypeStruct((PAD_N, C), jnp.float32),
)

@functools.cache
def _sc_mesh():
    return plsc.VectorSubcoreMesh(
        core_axis_name="c", subcore_axis_name="s",
        num_cores=NC, num_subcores=NS)


@functools.cache
def _build_dispatch():
    @functools.partial(
        pl.kernel,
        mesh=_sc_mesh(),
        out_type=[
            jax.ShapeDtypeStruct((PAD_N, C), jnp.float32),
            jax.ShapeDtypeStruct((PAD_N, WV), jnp.float32),
        ],
        scratch_types=[
            pltpu.VMEM((TPW,), jnp.int32),
            pltpu.VMEM((TPW, C), jnp.float32),
            pltpu.VMEM((TPW, WV), jnp.float32),
            pltpu.SemaphoreType.DMA,
            pltpu.SemaphoreType.DMA,
        ],
    )
    def dispatch(x_hbm, w_hbm, p_hbm, xs_hbm, ws_hbm, idx_v, rows_v, wv,
                 s1, s2):
        wid = lax.axis_index("s") * NC + lax.axis_index("c")
        basetok = wid * TPW
        pltpu.sync_copy(p_hbm.at[wid], idx_v)
        pltpu.sync_copy(x_hbm.at[pl.ds(basetok, TPW)], rows_v)
        pltpu.sync_copy(w_hbm.at[pl.ds(basetok, TPW)], wv)
        cp1 = pltpu.async_copy(rows_v, xs_hbm.at[idx_v], s1)
        cp2 = pltpu.async_copy(wv, ws_hbm.at[idx_v], s2)
        cp1.wait()
        cp2.wait()

    return dispatch


@functools.cache
def _build_collect():
    @functools.partial(
        pl.kernel,
        mesh=_sc_mesh(),
        out_type=jax.ShapeDtypeStruct((N, C), jnp.float32),
        scratch_types=[
            pltpu.VMEM((TPW,), jnp.int32),
            pltpu.VMEM((TPW, C), jnp.float32),
            pltpu.SemaphoreType.DMA,
        ],
    )
    def collect(ys_hbm, p_hbm, out_hbm, idx_v, rows_v, sem):
        wid = lax.axis_index("s") * NC + lax.axis_index("c")
        basetok = wid * TPW
        pltpu.sync_copy(p_hbm.at[wid], idx_v)
        pltpu.async_copy(ys_hbm.at[idx_v], rows_v, sem).wait()
        pltpu.sync_copy(rows_v, out_hbm.at[pl.ds(basetok, TPW)])

    return collect


def kernel(x, Wr, br, We, be):
    x2 = x.reshape(N, C)
    p2, bexp2, w16, aux = _router(x2, Wr, br.reshape(1, E))
    xs, ws = _build_dispatch()(x2, w16, p2)
    ys = _expert_mm(bexp2.reshape(NB), xs, ws, We, be.reshape(E, 1, C))
    out = _build_collect()(ys, p2)
    return out.reshape(B, T, C), aux[0, 0]
